# Initial kernel scaffold; baseline (speedup 1.0000x reference)
import jax
import jax.numpy as jnp
from jax.experimental import pallas as pl


def kernel(x, pe):
    raise NotImplementedError("write your pallas kernel here")


# speedup vs baseline: 1.7280x; 1.7280x over previous
import jax
import jax.numpy as jnp
from jax.experimental import pallas as pl

BLOCK_S = 512


def _add_kernel(x_ref, pe_ref, out_ref):
    out_ref[...] = x_ref[...] + pe_ref[...][None, :, :]


def kernel(x, pe):
    batch, seq_len, dim = x.shape
    grid = (seq_len // BLOCK_S,)
    return pl.pallas_call(
        _add_kernel,
        grid=grid,
        in_specs=[
            pl.BlockSpec((batch, BLOCK_S, dim), lambda i: (0, i, 0)),
            pl.BlockSpec((BLOCK_S, dim), lambda i: (i, 0)),
        ],
        out_specs=pl.BlockSpec((batch, BLOCK_S, dim), lambda i: (0, i, 0)),
        out_shape=jax.ShapeDtypeStruct((batch, seq_len, dim), x.dtype),
    )(x, pe[:seq_len])
